# Initial kernel scaffold; baseline (speedup 1.0000x reference)
import jax
import jax.numpy as jnp
from jax.experimental import pallas as pl


def kernel(node_weight, edge_index, edge_weight, W1, b1, W2, b2, W3, b3, gamma, beta):
    raise NotImplementedError("write your pallas kernel here")


# speedup vs baseline: 6.2725x; 6.2725x over previous
import functools

import jax
import jax.numpy as jnp
from jax import lax
from jax.experimental import pallas as pl
from jax.experimental.pallas import tpu as pltpu
from jax.experimental.pallas import tpu_sc as plsc

N = 10000
NP = 10240
D = 128
E = 320000
NC = 2
NS = 16
L = 16
NW = NC * NS
EPW = E // NW
K = 80
NCHUNK = EPW // K
RPT = NP // NS
ZR = 128
BN_EPS = 1e-5

_mesh = plsc.VectorSubcoreMesh(core_axis_name="c", subcore_axis_name="s")


def _splat(vec, e):
    return lax.gather(
        vec, jnp.full((L, 1), e, jnp.int32),
        lax.GatherDimensionNumbers(offset_dims=(), collapsed_slice_dims=(0,),
                                   start_index_map=(0,)),
        (1,), mode=lax.GatherScatterMode.PROMISE_IN_BOUNDS)


@functools.partial(
    pl.kernel,
    out_type=(jax.ShapeDtypeStruct((NC * NP,), jnp.float32),
              jax.ShapeDtypeStruct((NC * NP,), jnp.float32)),
    mesh=_mesh,
    scratch_types=[
        pltpu.VMEM((NCHUNK, K), jnp.int32),
        pltpu.VMEM((NCHUNK, K), jnp.int32),
        pltpu.VMEM((K,), jnp.float32),
        pltpu.VMEM((1024,), jnp.float32),
        pltpu.VMEM_SHARED((NP,), jnp.float32),
        pltpu.VMEM_SHARED((NP,), jnp.float32),
    ])
def _degrees(src_hbm, dst_hbm, dego_out, degi_out,
             src_v, dst_v, ones_v, zero_v, dego_sh, degi_sh):
    c = lax.axis_index("c")
    s = lax.axis_index("s")
    wid = s * NC + c
    pltpu.sync_copy(src_hbm.at[wid], src_v)
    pltpu.sync_copy(dst_hbm.at[wid], dst_v)
    for g in range(K // L):
        ones_v[pl.ds(g * L, L)] = jnp.ones((L,), jnp.float32)
    def zfill(i, carry):
        zero_v[pl.ds(i * L, L)] = jnp.zeros((L,), jnp.float32)
        return carry
    lax.fori_loop(0, 1024 // L, zfill, 0)
    @pl.when(s < NP // 1024)
    def _():
        pltpu.sync_copy(zero_v, dego_sh.at[pl.ds(s * 1024, 1024)])
        pltpu.sync_copy(zero_v, degi_sh.at[pl.ds(s * 1024, 1024)])
    plsc.subcore_barrier()
    def body(j, carry):
        pltpu.sync_copy(ones_v, dego_sh.at[src_v.at[j]], add=True)
        pltpu.sync_copy(ones_v, degi_sh.at[dst_v.at[j]], add=True)
        return carry
    lax.fori_loop(0, NCHUNK, body, 0)
    plsc.subcore_barrier()
    @pl.when(s < NP // 1024)
    def _():
        pltpu.sync_copy(dego_sh.at[pl.ds(s * 1024, 1024)],
                        dego_out.at[pl.ds(c * NP + s * 1024, 1024)])
        pltpu.sync_copy(degi_sh.at[pl.ds(s * 1024, 1024)],
                        degi_out.at[pl.ds(c * NP + s * 1024, 1024)])


NBLK = 5
BLK = NCHUNK // NBLK


@functools.partial(
    pl.kernel,
    out_type=jax.ShapeDtypeStruct((NC, NP, D), jnp.float32),
    mesh=_mesh,
    scratch_types=[
        pltpu.VMEM((BLK, K), jnp.int32),
        pltpu.VMEM((BLK, K), jnp.int32),
        pltpu.VMEM((BLK, K), jnp.float32),
        pltpu.VMEM((K, D), jnp.float32),
        pltpu.VMEM_SHARED((NP, D), jnp.float32),
    ])
def _edge_pass(m_hbm, src_hbm, dst_hbm, ew_hbm, out_hbm,
               sidx_v, didx_v, ewb_v, rows_v, agg_sh):
    c = lax.axis_index("c")
    s = lax.axis_index("s")
    wid = s * NC + c
    def zfill(i, carry):
        for dd in range(D // L):
            rows_v[i, pl.ds(dd * L, L)] = jnp.zeros((L,), jnp.float32)
        return carry
    lax.fori_loop(0, K, zfill, 0)
    for t in range(RPT // K):
        pltpu.sync_copy(rows_v, agg_sh.at[pl.ds(s * RPT + t * K, K)])
    plsc.subcore_barrier()
    def blk(b, carry):
        pltpu.sync_copy(src_hbm.at[wid, b], sidx_v)
        pltpu.sync_copy(dst_hbm.at[wid, b], didx_v)
        pltpu.sync_copy(ew_hbm.at[wid, b], ewb_v)
        def body(j, carry2):
            pltpu.sync_copy(m_hbm.at[sidx_v.at[j]], rows_v)
            for g in range(K // L):
                ewg = ewb_v[j, pl.ds(g * L, L)]
                for e in range(L):
                    scale = _splat(ewg, e)
                    row = g * L + e
                    for dd in range(D // L):
                        rows_v[row, pl.ds(dd * L, L)] = (
                            rows_v[row, pl.ds(dd * L, L)] * scale)
            pltpu.sync_copy(rows_v, agg_sh.at[didx_v.at[j]], add=True)
            return carry2
        lax.fori_loop(0, BLK, body, 0)
        return carry
    lax.fori_loop(0, NBLK, blk, 0)
    plsc.subcore_barrier()
    pltpu.sync_copy(agg_sh.at[pl.ds(s * RPT, RPT)],
                    out_hbm.at[c, pl.ds(s * RPT, RPT)])


def _prep_body(x_ref, dgo_ref, dgi_ref, m1_ref, nsrc_ref, ndst_ref):
    nsrc = lax.rsqrt(jnp.clip(dgo_ref[0] + dgo_ref[1], 1.0))
    ndst = lax.rsqrt(jnp.clip(dgi_ref[0] + dgi_ref[1], 1.0))
    m1_ref[...] = x_ref[...] * nsrc
    nsrc_ref[...] = nsrc
    ndst_ref[...] = ndst


_prep = pl.pallas_call(
    _prep_body,
    out_shape=(jax.ShapeDtypeStruct((NP, D), jnp.float32),
               jax.ShapeDtypeStruct((NP, 1), jnp.float32),
               jax.ShapeDtypeStruct((NP, 1), jnp.float32)))


def _dense_body(aggp_ref, ndst_ref, nsrc_ref, w_ref, b_ref,
                g_ref, bt_ref, hbn_ref, hact_ref):
    agg = (aggp_ref[0] + aggp_ref[1]) * ndst_ref[...]
    h = jnp.dot(agg, w_ref[...], preferred_element_type=jnp.float32)
    h = h + b_ref[...]
    h = h * (g_ref[...] / jnp.sqrt(1.0 + BN_EPS)) + bt_ref[...]
    hbn_ref[...] = h
    h = jnp.where(h > 0, h, jnp.exp(jnp.minimum(h, 0.0)) - 1.0)
    hact_ref[...] = h * nsrc_ref[...]


_dense = pl.pallas_call(
    _dense_body,
    out_shape=(jax.ShapeDtypeStruct((NP, D), jnp.float32),
               jax.ShapeDtypeStruct((NP, D), jnp.float32)))


def kernel(node_weight, edge_index, edge_weight,
           W1, b1, W2, b2, W3, b3, gamma, beta):
    src = edge_index[0].reshape(NW, NCHUNK, K)
    dst = edge_index[1].reshape(NW, NCHUNK, K)
    ew = edge_weight.reshape(NW, NCHUNK, K)
    src4 = edge_index[0].reshape(NW, NBLK, BLK, K)
    dst4 = edge_index[1].reshape(NW, NBLK, BLK, K)
    ew4 = edge_weight.reshape(NW, NBLK, BLK, K)
    dego, degi = _degrees(src, dst)
    xp = jnp.pad(node_weight, ((0, NP - N), (0, 0)))
    m, nsrc, ndst = _prep(xp,
                          dego.reshape(NC, NP, 1), degi.reshape(NC, NP, 1))
    gr = gamma.reshape(1, D)
    br = beta.reshape(1, D)
    Ws = jnp.stack([W1, W2, W3])
    bs = jnp.stack([b1.reshape(1, D), b2.reshape(1, D), b3.reshape(1, D)])

    def layer(carry, wb):
        m_cur, _ = carry
        W, b = wb
        aggp = _edge_pass(m_cur, src4, dst4, ew4)
        hbn, hact = _dense(aggp, ndst, nsrc, W, b, gr, br)
        return (hact, hbn), None

    (_, hbn_final), _ = lax.scan(layer, (m, m), (Ws, bs))
    return hbn_final[:N]


# speedup vs baseline: 8.1572x; 1.3005x over previous
import functools

import jax
import jax.numpy as jnp
from jax import lax
from jax.experimental import pallas as pl
from jax.experimental.pallas import tpu as pltpu
from jax.experimental.pallas import tpu_sc as plsc

N = 10000
NP = 10240
D = 128
E = 320000
NC = 2
NS = 16
L = 16
NW = NC * NS
EPW = E // NW
K = 80
NCHUNK = EPW // K
RPT = NP // NS
ZR = 128
BN_EPS = 1e-5

_mesh = plsc.VectorSubcoreMesh(core_axis_name="c", subcore_axis_name="s")


def _splat(vec, e):
    return lax.gather(
        vec, jnp.full((L, 1), e, jnp.int32),
        lax.GatherDimensionNumbers(offset_dims=(), collapsed_slice_dims=(0,),
                                   start_index_map=(0,)),
        (1,), mode=lax.GatherScatterMode.PROMISE_IN_BOUNDS)


@functools.partial(
    pl.kernel,
    out_type=(jax.ShapeDtypeStruct((NC * NP,), jnp.float32),
              jax.ShapeDtypeStruct((NC * NP,), jnp.float32)),
    mesh=_mesh,
    scratch_types=[
        pltpu.VMEM((NCHUNK, K), jnp.int32),
        pltpu.VMEM((NCHUNK, K), jnp.int32),
        pltpu.VMEM((K,), jnp.float32),
        pltpu.VMEM((1024,), jnp.float32),
        pltpu.VMEM_SHARED((NP,), jnp.float32),
        pltpu.VMEM_SHARED((NP,), jnp.float32),
    ])
def _degrees(src_hbm, dst_hbm, dego_out, degi_out,
             src_v, dst_v, ones_v, zero_v, dego_sh, degi_sh):
    c = lax.axis_index("c")
    s = lax.axis_index("s")
    wid = s * NC + c
    pltpu.sync_copy(src_hbm.at[wid], src_v)
    pltpu.sync_copy(dst_hbm.at[wid], dst_v)
    for g in range(K // L):
        ones_v[pl.ds(g * L, L)] = jnp.ones((L,), jnp.float32)
    def zfill(i, carry):
        zero_v[pl.ds(i * L, L)] = jnp.zeros((L,), jnp.float32)
        return carry
    lax.fori_loop(0, 1024 // L, zfill, 0)
    @pl.when(s < NP // 1024)
    def _():
        pltpu.sync_copy(zero_v, dego_sh.at[pl.ds(s * 1024, 1024)])
        pltpu.sync_copy(zero_v, degi_sh.at[pl.ds(s * 1024, 1024)])
    plsc.subcore_barrier()
    def body(j, carry):
        pltpu.sync_copy(ones_v, dego_sh.at[src_v.at[j]], add=True)
        pltpu.sync_copy(ones_v, degi_sh.at[dst_v.at[j]], add=True)
        return carry
    lax.fori_loop(0, NCHUNK, body, 0)
    plsc.subcore_barrier()
    @pl.when(s < NP // 1024)
    def _():
        pltpu.sync_copy(dego_sh.at[pl.ds(s * 1024, 1024)],
                        dego_out.at[pl.ds(c * NP + s * 1024, 1024)])
        pltpu.sync_copy(degi_sh.at[pl.ds(s * 1024, 1024)],
                        degi_out.at[pl.ds(c * NP + s * 1024, 1024)])


NBLK = 5
BLK = NCHUNK // NBLK


@functools.partial(
    pl.kernel,
    out_type=jax.ShapeDtypeStruct((NC, NP, D), jnp.float32),
    mesh=_mesh,
    scratch_types=[
        pltpu.VMEM((BLK, K), jnp.int32),
        pltpu.VMEM((BLK, K), jnp.int32),
        pltpu.VMEM((BLK, K), jnp.float32),
        pltpu.VMEM((K, D), jnp.float32),
        pltpu.VMEM((K, D), jnp.float32),
        pltpu.SemaphoreType.DMA,
        pltpu.SemaphoreType.DMA,
        pltpu.VMEM_SHARED((NP, D), jnp.float32),
    ])
def _edge_pass(m_hbm, src_hbm, dst_hbm, ew_hbm, out_hbm,
               sidx_v, didx_v, ewb_v, rows0_v, rows1_v, sem0, sem1, agg_sh):
    c = lax.axis_index("c")
    s = lax.axis_index("s")
    wid = s * NC + c
    def zfill(i, carry):
        for dd in range(D // L):
            rows0_v[i, pl.ds(dd * L, L)] = jnp.zeros((L,), jnp.float32)
        return carry
    lax.fori_loop(0, K, zfill, 0)
    for t in range(RPT // K):
        pltpu.sync_copy(rows0_v, agg_sh.at[pl.ds(s * RPT + t * K, K)])
    plsc.subcore_barrier()

    def scale_scatter(rows_v, j):
        for g in range(K // L):
            ewg = ewb_v[j, pl.ds(g * L, L)]
            for e in range(L):
                scale = _splat(ewg, e)
                row = g * L + e
                for dd in range(D // L):
                    rows_v[row, pl.ds(dd * L, L)] = (
                        rows_v[row, pl.ds(dd * L, L)] * scale)
        pltpu.sync_copy(rows_v, agg_sh.at[didx_v.at[j]], add=True)

    def blk(b, carry):
        pltpu.sync_copy(src_hbm.at[wid, b], sidx_v)
        pltpu.sync_copy(dst_hbm.at[wid, b], didx_v)
        pltpu.sync_copy(ew_hbm.at[wid, b], ewb_v)
        pltpu.async_copy(m_hbm.at[sidx_v.at[0]], rows0_v, sem0)

        def body(i, carry2):
            j0 = 2 * i
            pltpu.make_async_copy(m_hbm.at[pl.ds(0, K)], rows0_v, sem0).wait()
            pltpu.async_copy(m_hbm.at[sidx_v.at[j0 + 1]], rows1_v, sem1)
            scale_scatter(rows0_v, j0)
            pltpu.make_async_copy(m_hbm.at[pl.ds(0, K)], rows1_v, sem1).wait()
            pltpu.async_copy(m_hbm.at[sidx_v.at[j0 + 2]], rows0_v, sem0)
            scale_scatter(rows1_v, j0 + 1)
            return carry2
        lax.fori_loop(0, BLK // 2, body, 0)
        pltpu.make_async_copy(m_hbm.at[pl.ds(0, K)], rows0_v, sem0).wait()
        scale_scatter(rows0_v, BLK - 1)
        return carry
    lax.fori_loop(0, NBLK, blk, 0)
    plsc.subcore_barrier()
    pltpu.sync_copy(agg_sh.at[pl.ds(s * RPT, RPT)],
                    out_hbm.at[c, pl.ds(s * RPT, RPT)])


def _prep_body(x_ref, dgo_ref, dgi_ref, m1_ref, nsrc_ref, ndst_ref):
    nsrc = lax.rsqrt(jnp.clip(dgo_ref[0] + dgo_ref[1], 1.0))
    ndst = lax.rsqrt(jnp.clip(dgi_ref[0] + dgi_ref[1], 1.0))
    m1_ref[...] = x_ref[...] * nsrc
    nsrc_ref[...] = nsrc
    ndst_ref[...] = ndst


_prep = pl.pallas_call(
    _prep_body,
    out_shape=(jax.ShapeDtypeStruct((NP, D), jnp.float32),
               jax.ShapeDtypeStruct((NP, 1), jnp.float32),
               jax.ShapeDtypeStruct((NP, 1), jnp.float32)))


def _dense_body(aggp_ref, ndst_ref, nsrc_ref, w_ref, b_ref,
                g_ref, bt_ref, hbn_ref, hact_ref):
    agg = (aggp_ref[0] + aggp_ref[1]) * ndst_ref[...]
    h = jnp.dot(agg, w_ref[...], preferred_element_type=jnp.float32)
    h = h + b_ref[...]
    h = h * (g_ref[...] / jnp.sqrt(1.0 + BN_EPS)) + bt_ref[...]
    hbn_ref[...] = h
    h = jnp.where(h > 0, h, jnp.exp(jnp.minimum(h, 0.0)) - 1.0)
    hact_ref[...] = h * nsrc_ref[...]


_dense = pl.pallas_call(
    _dense_body,
    out_shape=(jax.ShapeDtypeStruct((NP, D), jnp.float32),
               jax.ShapeDtypeStruct((NP, D), jnp.float32)))


def kernel(node_weight, edge_index, edge_weight,
           W1, b1, W2, b2, W3, b3, gamma, beta):
    src = edge_index[0].reshape(NW, NCHUNK, K)
    dst = edge_index[1].reshape(NW, NCHUNK, K)
    ew = edge_weight.reshape(NW, NCHUNK, K)
    src4 = edge_index[0].reshape(NW, NBLK, BLK, K)
    dst4 = edge_index[1].reshape(NW, NBLK, BLK, K)
    ew4 = edge_weight.reshape(NW, NBLK, BLK, K)
    dego, degi = _degrees(src, dst)
    xp = jnp.pad(node_weight, ((0, NP - N), (0, 0)))
    m, nsrc, ndst = _prep(xp,
                          dego.reshape(NC, NP, 1), degi.reshape(NC, NP, 1))
    gr = gamma.reshape(1, D)
    br = beta.reshape(1, D)
    Ws = jnp.stack([W1, W2, W3])
    bs = jnp.stack([b1.reshape(1, D), b2.reshape(1, D), b3.reshape(1, D)])

    def layer(carry, wb):
        m_cur, _ = carry
        W, b = wb
        aggp = _edge_pass(m_cur, src4, dst4, ew4)
        hbn, hact = _dense(aggp, ndst, nsrc, W, b, gr, br)
        return (hact, hbn), None

    (_, hbn_final), _ = lax.scan(layer, (m, m), (Ws, bs))
    return hbn_final[:N]
